# baseline (speedup 1.0000x reference)
import functools

import jax
import jax.numpy as jnp
from jax import lax
from jax.experimental import pallas as pl
from jax.experimental.pallas import tpu as pltpu
from jax.experimental.pallas import tpu_sc as plsc

LANES = 16
QUARTER = 128


@functools.lru_cache(maxsize=None)
def _make_kernel(B, D):
    info = plsc.get_sparse_core_info()
    NC, NS = info.num_cores, info.num_subcores
    NW = NC * NS
    assert B % NW == 0 and D % LANES == 0
    b_per_w = B // NW
    n_q = b_per_w // QUARTER
    pack = 128 // D
    mesh = plsc.VectorSubcoreMesh(core_axis_name="c", subcore_axis_name="s")

    @functools.partial(
        pl.kernel,
        mesh=mesh,
        out_type=jax.ShapeDtypeStruct((B, D), jnp.float32),
        scratch_types=[
            pltpu.VMEM((b_per_w,), jnp.int32),
            pltpu.VMEM((b_per_w,), jnp.int32),
            pltpu.VMEM((b_per_w,), jnp.int32),
            pltpu.VMEM((b_per_w,), jnp.int32),
            pltpu.VMEM((b_per_w,), jnp.int32),
            pltpu.VMEM((b_per_w,), jnp.int32),
            pltpu.VMEM((QUARTER, 128), jnp.float32),
            pltpu.VMEM((QUARTER, 128), jnp.float32),
            pltpu.VMEM((b_per_w, D), jnp.float32),
            pltpu.SemaphoreType.DMA,
            pltpu.SemaphoreType.DMA,
        ],
    )
    def gmf(uids, sids, upk, spk, out, uidx, sidx, urow, srow, uoff, soff,
            ublk, sblk, prod, sem_u, sem_s):
        wid = lax.axis_index("s") * NC + lax.axis_index("c")
        base = wid * b_per_w
        pltpu.sync_copy(uids.at[pl.ds(base, b_per_w)], uidx)
        pltpu.sync_copy(sids.at[pl.ds(base, b_per_w)], sidx)

        def prep(g, carry):
            sl = pl.ds(g * LANES, LANES)
            uv = uidx[sl]
            sv = sidx[sl]
            urow[sl] = uv >> 2
            srow[sl] = sv >> 2
            uoff[sl] = (uv & (pack - 1)) * D
            soff[sl] = (sv & (pack - 1)) * D
            return carry

        lax.fori_loop(0, b_per_w // LANES, prep, 0)

        for q in range(n_q):
            lo = q * QUARTER
            cu = pltpu.async_copy(
                upk.at[urow.at[pl.ds(lo, QUARTER)]], ublk, sem_u)
            cs = pltpu.async_copy(
                spk.at[srow.at[pl.ds(lo, QUARTER)]], sblk, sem_s)
            cu.wait()
            cs.wait()

            def body(g, carry):
                gb = g * LANES
                uo = uoff[pl.ds(lo + gb, LANES)]
                so = soff[pl.ds(lo + gb, LANES)]
                for j in range(LANES):
                    i = gb + j
                    for k in range(D // LANES):
                        sl = pl.ds(k * LANES, LANES)
                        prod[lo + i, sl] = (
                            ublk[i, pl.ds(uo[j] + k * LANES, LANES)]
                            * sblk[i, pl.ds(so[j] + k * LANES, LANES)])
                return carry

            lax.fori_loop(0, QUARTER // LANES, body, 0)

        pltpu.sync_copy(prod, out.at[pl.ds(base, b_per_w)])

    return gmf


def kernel(users_ids, services_ids, user_table, service_table):
    B, = users_ids.shape
    V, D = user_table.shape
    gmf = _make_kernel(B, D)
    pack = 128 // D
    return gmf(
        users_ids.astype(jnp.int32),
        services_ids.astype(jnp.int32),
        user_table.reshape(V // pack, 128),
        service_table.reshape(V // pack, 128),
    )


# speedup vs baseline: 2.5342x; 2.5342x over previous
import functools

import jax
import jax.numpy as jnp
from jax import lax
from jax.experimental import pallas as pl
from jax.experimental.pallas import tpu as pltpu
from jax.experimental.pallas import tpu_sc as plsc

LANES = 16
CHUNK = 64


@functools.lru_cache(maxsize=None)
def _make_kernel(B, D):
    info = plsc.get_sparse_core_info()
    NC, NS = info.num_cores, info.num_subcores
    NW = NC * NS
    assert B % NW == 0 and D % LANES == 0
    b_per_w = B // NW
    mesh = plsc.VectorSubcoreMesh(core_axis_name="c", subcore_axis_name="s")

    @functools.partial(
        pl.kernel,
        mesh=mesh,
        out_type=jax.ShapeDtypeStruct((B, D), jnp.float32),
        scratch_types=[
            pltpu.VMEM((b_per_w,), jnp.int32),
            pltpu.VMEM((b_per_w,), jnp.int32),
            pltpu.VMEM((CHUNK, 1, D), jnp.float32),
            pltpu.VMEM((CHUNK, 1, D), jnp.float32),
            pltpu.VMEM((b_per_w, D), jnp.float32),
            pltpu.SemaphoreType.DMA,
            pltpu.SemaphoreType.DMA,
        ],
    )
    def gmf(uids, sids, utab3, stab3, out, uidx, sidx,
            ublk, sblk, prod, sem_u, sem_s):
        wid = lax.axis_index("s") * NC + lax.axis_index("c")
        base = wid * b_per_w
        pltpu.sync_copy(uids.at[pl.ds(base, b_per_w)], uidx)
        pltpu.sync_copy(sids.at[pl.ds(base, b_per_w)], sidx)

        def chunk_body(c, carry):
            lo = c * CHUNK
            uvecs = [uidx[pl.ds(lo + g * LANES, LANES)]
                     for g in range(CHUNK // LANES)]
            svecs = [sidx[pl.ds(lo + g * LANES, LANES)]
                     for g in range(CHUNK // LANES)]
            for g in range(CHUNK // LANES):
                for j in range(LANES):
                    i = g * LANES + j
                    pltpu.make_async_copy(
                        utab3.at[uvecs[g][j]], ublk.at[i], sem_u).start()
                    pltpu.make_async_copy(
                        stab3.at[svecs[g][j]], sblk.at[i], sem_s).start()
            for i in range(CHUNK):
                pltpu.make_async_copy(
                    utab3.at[0], ublk.at[i], sem_u).wait()
                pltpu.make_async_copy(
                    stab3.at[0], sblk.at[i], sem_s).wait()

            def body(g, carry2):
                gb = g * LANES
                for j in range(LANES):
                    i = gb + j
                    for k in range(D // LANES):
                        sl = pl.ds(k * LANES, LANES)
                        prod[lo + i, sl] = ublk[i, 0, sl] * sblk[i, 0, sl]
                return carry2

            lax.fori_loop(0, CHUNK // LANES, body, 0)
            return carry

        lax.fori_loop(0, b_per_w // CHUNK, chunk_body, 0)
        pltpu.sync_copy(prod, out.at[pl.ds(base, b_per_w)])

    return gmf


def kernel(users_ids, services_ids, user_table, service_table):
    B, = users_ids.shape
    V, D = user_table.shape
    gmf = _make_kernel(B, D)
    return gmf(
        users_ids.astype(jnp.int32),
        services_ids.astype(jnp.int32),
        user_table.reshape(V, 1, D),
        service_table.reshape(V, 1, D),
    )
